# baseline (speedup 1.0000x reference)
import functools

import jax
import jax.numpy as jnp
from jax import lax
from jax.experimental import pallas as pl
from jax.experimental.pallas import tpu as pltpu
from jax.experimental.pallas import tpu_sc as plsc

VOCAB = 30522
LANG_DIM = 768
OUT_DIM = 1024
BATCH = 4096
SEQ = 20
TOKENS = BATCH * SEQ


BM = 1024
GRID_M = (VOCAB + BM - 1) // BM


def _proj_body(t_ref, w_ref, b_ref, m_ref, p_ref, pm_ref):
    p_ref[...] = (
        jnp.dot(
            t_ref[...].astype(jnp.bfloat16),
            w_ref[...].astype(jnp.bfloat16),
            preferred_element_type=jnp.float32,
        )
        + b_ref[...]
    )

    @pl.when(pl.program_id(0) == 0)
    def _():
        pm_ref[...] = (m_ref[...] == 0).astype(jnp.int32)


def _project(emb_table, W, b2d, attention_mask):
    return pl.pallas_call(
        _proj_body,
        grid=(GRID_M,),
        in_specs=[
            pl.BlockSpec((BM, LANG_DIM), lambda i: (i, 0)),
            pl.BlockSpec((LANG_DIM, OUT_DIM), lambda i: (0, 0)),
            pl.BlockSpec((1, OUT_DIM), lambda i: (0, 0)),
            pl.BlockSpec((BATCH, SEQ), lambda i: (0, 0)),
        ],
        out_specs=[
            pl.BlockSpec((BM, OUT_DIM), lambda i: (i, 0)),
            pl.BlockSpec((BATCH, SEQ), lambda i: (0, 0)),
        ],
        out_shape=[
            jax.ShapeDtypeStruct((VOCAB, OUT_DIM), jnp.float32),
            jax.ShapeDtypeStruct((BATCH, SEQ), jnp.int32),
        ],
    )(emb_table, W, b2d, attention_mask)


NC, NS = 2, 16
NW = NC * NS
BPW = TOKENS // NW
CHUNK = 40
NCHUNK = BPW // CHUNK


@functools.lru_cache(maxsize=1)
def _make_sc_gather():
    mesh = plsc.VectorSubcoreMesh(core_axis_name="c", subcore_axis_name="s")

    @functools.partial(
        pl.kernel,
        mesh=mesh,
        out_type=jax.ShapeDtypeStruct((TOKENS, OUT_DIM), jnp.float32),
        scratch_types=[
            pltpu.VMEM((BPW,), jnp.int32),
            pltpu.VMEM((CHUNK, OUT_DIM), jnp.float32),
            pltpu.SemaphoreType.DMA,
        ],
    )
    def _sc_gather(p_hbm, idx_hbm, out_hbm, idx_v, buf, sem):
        wid = lax.axis_index("s") * NC + lax.axis_index("c")
        base = wid * BPW
        pltpu.sync_copy(idx_hbm.at[pl.ds(base, BPW)], idx_v)

        def body(c, carry):
            pltpu.async_copy(
                p_hbm.at[idx_v.at[pl.ds(c * CHUNK, CHUNK)]], buf, sem
            ).wait()
            pltpu.sync_copy(buf, out_hbm.at[pl.ds(base + c * CHUNK, CHUNK)])
            return carry

        lax.fori_loop(0, NCHUNK, body, 0)

    return _sc_gather


def kernel(ref_expr_inds, attention_mask, emb_table, W, b):
    P, pm = _project(emb_table, W, b.reshape(1, OUT_DIM), attention_mask)
    idx = ref_expr_inds.reshape(TOKENS)
    out = _make_sc_gather()(P, idx)
    return out.reshape(BATCH, SEQ, OUT_DIM), pm.astype(jnp.bool_)


# speedup vs baseline: 1.0007x; 1.0007x over previous
import functools

import jax
import jax.numpy as jnp
from jax import lax
from jax.experimental import pallas as pl
from jax.experimental.pallas import tpu as pltpu
from jax.experimental.pallas import tpu_sc as plsc

VOCAB = 30522
LANG_DIM = 768
OUT_DIM = 1024
BATCH = 4096
SEQ = 20
TOKENS = BATCH * SEQ


BM = 1024
GRID_M = (VOCAB + BM - 1) // BM


def _proj_body(t_ref, w_ref, b_ref, m_ref, p_ref, pm_ref):
    p_ref[...] = (
        jnp.dot(
            t_ref[...].astype(jnp.bfloat16),
            w_ref[...].astype(jnp.bfloat16),
            preferred_element_type=jnp.float32,
        )
        + b_ref[...]
    )

    @pl.when(pl.program_id(0) == 0)
    def _():
        pm_ref[...] = (m_ref[...] == 0).astype(jnp.int32)


def _project(emb_table, W, b2d, attention_mask):
    return pl.pallas_call(
        _proj_body,
        grid=(GRID_M,),
        in_specs=[
            pl.BlockSpec((BM, LANG_DIM), lambda i: (i, 0)),
            pl.BlockSpec((LANG_DIM, OUT_DIM), lambda i: (0, 0)),
            pl.BlockSpec((1, OUT_DIM), lambda i: (0, 0)),
            pl.BlockSpec((BATCH, SEQ), lambda i: (0, 0)),
        ],
        out_specs=[
            pl.BlockSpec((BM, OUT_DIM), lambda i: (i, 0)),
            pl.BlockSpec((BATCH, SEQ), lambda i: (0, 0)),
        ],
        out_shape=[
            jax.ShapeDtypeStruct((VOCAB, OUT_DIM), jnp.float32),
            jax.ShapeDtypeStruct((BATCH, SEQ), jnp.int32),
        ],
    )(emb_table, W, b2d, attention_mask)


NC, NS = 2, 16
NW = NC * NS
BPW = TOKENS // NW
CHUNK = 40
NCHUNK = BPW // CHUNK


@functools.lru_cache(maxsize=1)
def _make_sc_gather():
    mesh = plsc.VectorSubcoreMesh(core_axis_name="c", subcore_axis_name="s")

    @functools.partial(
        pl.kernel,
        mesh=mesh,
        out_type=jax.ShapeDtypeStruct((TOKENS, OUT_DIM), jnp.float32),
        scratch_types=[
            pltpu.VMEM((BPW,), jnp.int32),
            pltpu.VMEM((CHUNK, OUT_DIM), jnp.float32),
            pltpu.SemaphoreType.DMA,
        ],
        compiler_params=pltpu.CompilerParams(use_tc_tiling_on_sc=True),
    )
    def _sc_gather(p_hbm, idx_hbm, out_hbm, idx_v, buf, sem):
        wid = lax.axis_index("s") * NC + lax.axis_index("c")
        base = wid * BPW
        pltpu.sync_copy(idx_hbm.at[pl.ds(base, BPW)], idx_v)

        def body(c, carry):
            pltpu.async_copy(
                p_hbm.at[idx_v.at[pl.ds(c * CHUNK, CHUNK)]], buf, sem
            ).wait()
            pltpu.sync_copy(buf, out_hbm.at[pl.ds(base + c * CHUNK, CHUNK)])
            return carry

        lax.fori_loop(0, NCHUNK, body, 0)

    return _sc_gather


def kernel(ref_expr_inds, attention_mask, emb_table, W, b):
    P, pm = _project(emb_table, W, b.reshape(1, OUT_DIM), attention_mask)
    idx = ref_expr_inds.reshape(TOKENS)
    out = _make_sc_gather()(P, idx)
    return out.reshape(BATCH, SEQ, OUT_DIM), pm.astype(jnp.bool_)
